# baseline (speedup 1.0000x reference)
import functools

import jax
import jax.numpy as jnp
from jax import lax
from jax.experimental import pallas as pl
from jax.experimental.pallas import tpu as pltpu
from jax.experimental.pallas import tpu_sc as plsc

N = 512
B = 128
L = N * (N + 1) // 2
NC = 2
NS = 16
NW = NC * NS
R = 8
W = 4352
BG = 8
HALF = N // 2
K = HALF // R
NSLOT = 4


def _start(j):
    return j * N - (j * (j - 1)) // 2 - j


_PERM_DNUMS = lax.GatherDimensionNumbers(
    offset_dims=(), collapsed_slice_dims=(0,), start_index_map=(0,)
)


def _perm16(vec, idx16):
    return lax.gather(
        vec, idx16[:, None], _PERM_DNUMS, (1,),
        mode=lax.GatherScatterMode.PROMISE_IN_BOUNDS,
    )


def _body(x_hbm, out_hbm, in0, in1, out_scr, sin0, sin1, out_sems):
    wid = lax.axis_index("s") * NC + lax.axis_index("c")
    bg = wid // 2
    h = wid % 2
    b0 = pl.multiple_of(bg * BG, 8)
    iota = lax.iota(jnp.int32, 16)
    in_bufs = (in0, in1)
    in_sems = (sin0, sin1)

    def in_slice(k):
        j0 = h * HALF + k * R
        s0 = _start(j0)
        c0 = jnp.minimum(s0 - s0 % 128, L - W)
        c0 = pl.multiple_of(c0, 128)
        return x_hbm.at[pl.ds(b0, 8), pl.ds(c0, W)], c0

    def out_slice(k, bl):
        j0 = pl.multiple_of(h * HALF + k * R, 8)
        return out_hbm.at[b0 + bl, pl.ds(j0, R), :]

    def start_in(k, d):
        src, _ = in_slice(k)
        pltpu.async_copy(src, in_bufs[d], in_sems[d])

    start_in(0, 0)

    @pl.loop(0, K, step=2)
    def block_loop(k0):
        for d in range(2):
            k = k0 + d
            in_buf = in_bufs[d]

            @pl.when(k + 1 < K)
            def _():
                start_in(k + 1, 1 - d)

            src, c0 = in_slice(k)
            pltpu.make_async_copy(src, in_buf, in_sems[d]).wait()
            j0 = h * HALF + k * R

            @pl.loop(0, BG)
            def batch_loop(bl):
                slot = bl % NSLOT
                slab = out_scr.at[slot]

                @pl.when((k >= 1) | (bl >= NSLOT))
                def _():
                    pltpu.make_async_copy(
                        slab, out_slice(k, bl), out_sems.at[slot]
                    ).wait()

                @pl.loop(0, R)
                def row_loop(r):
                    j = j0 + r
                    off = _start(j) - c0
                    s = off % 16
                    a0 = pl.multiple_of(off - s, 16)
                    perm = (iota + s) % 16
                    low = iota < (16 - s)
                    prev = _perm16(in_buf[bl, pl.ds(a0, 16)], perm)
                    for v in range(N // 16):
                        nxt = _perm16(in_buf[bl, pl.ds(a0 + v * 16 + 16, 16)], perm)
                        vals = jnp.where(low, prev, nxt)
                        col = iota + (v * 16)
                        slab[r, pl.ds(v * 16, 16)] = jnp.where(col >= j, vals, 0.0)
                        prev = nxt

                pltpu.async_copy(slab, out_slice(k, bl), out_sems.at[slot])

    for bl in range(BG - NSLOT, BG):
        slot = bl % NSLOT
        pltpu.make_async_copy(
            out_scr.at[slot], out_slice(K - 1, bl), out_sems.at[slot]
        ).wait()


@functools.partial(jax.jit, static_argnames=("interpret",))
def _unpack_triu(x, interpret=False):
    mesh = plsc.VectorSubcoreMesh(
        core_axis_name="c", subcore_axis_name="s", num_cores=NC, num_subcores=NS
    )
    f = pl.kernel(
        _body,
        out_type=jax.ShapeDtypeStruct((B, N, N), jnp.float32),
        mesh=mesh,
        scratch_types=[
            pltpu.VMEM((BG, W), jnp.float32),
            pltpu.VMEM((BG, W), jnp.float32),
            pltpu.VMEM((NSLOT, R, N), jnp.float32),
            pltpu.SemaphoreType.DMA,
            pltpu.SemaphoreType.DMA,
            pltpu.SemaphoreType.DMA((NSLOT,)),
        ],
        compiler_params=pltpu.CompilerParams(use_tc_tiling_on_sc=True),
        interpret=interpret,
    )
    return f(x)


def kernel(x, idx):
    return _unpack_triu(x)


# speedup vs baseline: 1.4119x; 1.4119x over previous
import functools

import jax
import jax.numpy as jnp
from jax import lax
from jax.experimental import pallas as pl
from jax.experimental.pallas import tpu as pltpu
from jax.experimental.pallas import tpu_sc as plsc

N = 512
B = 128
L = N * (N + 1) // 2
NC = 2
NS = 16
NW = NC * NS
R = 8
BG = 8
K = 32
NSLOT = 8
WMAX = 4224
TIER_W = (4224, 3328, 2432, 1536)


def _start(j):
    return j * N - (j * (j - 1)) // 2 - j


_PERM_DNUMS = lax.GatherDimensionNumbers(
    offset_dims=(), collapsed_slice_dims=(0,), start_index_map=(0,)
)


def _perm16(vec, idx16):
    return lax.gather(
        vec, idx16[:, None], _PERM_DNUMS, (1,),
        mode=lax.GatherScatterMode.PROMISE_IN_BOUNDS,
    )


def _body(x_hbm, out_hbm, in0, in1, out_scr, sin0, sin1, out_sems):
    wid = lax.axis_index("s") * NC + lax.axis_index("c")
    bg = wid // 2
    h = wid % 2
    b0 = pl.multiple_of(bg * BG, 8)
    iota = lax.iota(jnp.int32, 16)
    in_bufs = (in0, in1)
    in_sems = (sin0, sin1)

    def j0_of(k):
        return 16 * k + 8 * h

    def in_slice(k, w):
        s0 = _start(j0_of(k))
        c0 = jnp.minimum(s0 - s0 % 128, L - w)
        c0 = pl.multiple_of(c0, 128)
        return x_hbm.at[pl.ds(b0, 8), pl.ds(c0, w)], c0

    def out_slice(k, bl):
        j0 = pl.multiple_of(j0_of(k), 8)
        return out_hbm.at[b0 + bl, pl.ds(j0, R), :]

    def start_in(k, d, w):
        src, _ = in_slice(k, w)
        pltpu.async_copy(src, in_bufs[d].at[:, pl.ds(0, w)], in_sems[d])

    start_in(0, 0, TIER_W[0])

    for t in range(4):
        wt = TIER_W[t]

        @pl.loop(8 * t, 8 * t + 8, step=2)
        def block_loop(k0):
            for d in range(2):
                k = k0 + d
                in_buf = in_bufs[d]

                @pl.when(k + 1 < 8 * t + 8)
                def _():
                    start_in(k + 1, 1 - d, wt)

                src, c0 = in_slice(k, wt)
                pltpu.make_async_copy(src, in_buf.at[:, pl.ds(0, wt)], in_sems[d]).wait()
                j0 = j0_of(k)

                @pl.loop(0, BG)
                def batch_loop(bl):
                    slot = bl % NSLOT
                    slab = out_scr.at[slot]

                    @pl.when(k >= 1)
                    def _():
                        pltpu.make_async_copy(
                            slab, out_slice(k, bl), out_sems.at[slot]
                        ).wait()

                    @pl.loop(0, R)
                    def row_loop(r):
                        j = j0 + r
                        off = _start(j) - c0
                        s = off % 16
                        a0 = pl.multiple_of(off - s, 16)
                        perm = (iota + s) % 16
                        low = iota < (16 - s)
                        prev = _perm16(in_buf[bl, pl.ds(a0, 16)], perm)
                        for v in range(N // 16):
                            nxt = _perm16(in_buf[bl, pl.ds(a0 + v * 16 + 16, 16)], perm)
                            vals = jnp.where(low, prev, nxt)
                            col = iota + (v * 16)
                            slab[r, pl.ds(v * 16, 16)] = jnp.where(col >= j, vals, 0.0)
                            prev = nxt

                    pltpu.async_copy(slab, out_slice(k, bl), out_sems.at[slot])

        if t < 3:
            start_in(8 * (t + 1), 0, TIER_W[t + 1])

    for bl in range(BG):
        slot = bl % NSLOT
        pltpu.make_async_copy(
            out_scr.at[slot], out_slice(K - 1, bl), out_sems.at[slot]
        ).wait()


@functools.partial(jax.jit, static_argnames=("interpret",))
def _unpack_triu(x, interpret=False):
    mesh = plsc.VectorSubcoreMesh(
        core_axis_name="c", subcore_axis_name="s", num_cores=NC, num_subcores=NS
    )
    f = pl.kernel(
        _body,
        out_type=jax.ShapeDtypeStruct((B, N, N), jnp.float32),
        mesh=mesh,
        scratch_types=[
            pltpu.VMEM((BG, WMAX + 16), jnp.float32),
            pltpu.VMEM((BG, WMAX + 16), jnp.float32),
            pltpu.VMEM((NSLOT, R, N), jnp.float32),
            pltpu.SemaphoreType.DMA,
            pltpu.SemaphoreType.DMA,
            pltpu.SemaphoreType.DMA((NSLOT,)),
        ],
        compiler_params=pltpu.CompilerParams(use_tc_tiling_on_sc=True),
        interpret=interpret,
    )
    return f(x)


def kernel(x, idx):
    return _unpack_triu(x)


# speedup vs baseline: 1.4456x; 1.0239x over previous
import functools

import jax
import jax.numpy as jnp
from jax import lax
from jax.experimental import pallas as pl
from jax.experimental.pallas import tpu as pltpu
from jax.experimental.pallas import tpu_sc as plsc

N = 512
B = 128
L = N * (N + 1) // 2
NC = 2
NS = 16
NW = NC * NS
R = 8
BG = 8
K = 32
NSLOT = 8
WMAX = 4224
TIER_W = (4224, 3328, 2432, 1536)


def _start(j):
    return j * N - (j * (j - 1)) // 2 - j


_PERM_DNUMS = lax.GatherDimensionNumbers(
    offset_dims=(), collapsed_slice_dims=(0,), start_index_map=(0,)
)


def _perm16(vec, idx16):
    return lax.gather(
        vec, idx16[:, None], _PERM_DNUMS, (1,),
        mode=lax.GatherScatterMode.PROMISE_IN_BOUNDS,
    )


def _body(x_hbm, out_hbm, in0, in1, out_scr, zeros, sin0, sin1, out_sems):
    wid = lax.axis_index("s") * NC + lax.axis_index("c")
    bg = wid // 2
    h = wid % 2
    b0 = pl.multiple_of(bg * BG, 8)
    iota = lax.iota(jnp.int32, 16)
    in_bufs = (in0, in1)
    in_sems = (sin0, sin1)
    zvec = jnp.zeros((16,), jnp.float32)

    @pl.loop(0, 8)
    def zero_init(r):
        for v in range(8):
            zeros[r, pl.ds(v * 16, 16)] = zvec

    def j0_of(k):
        return 16 * k + 8 * h

    def in_slice(k, w):
        s0 = _start(j0_of(k))
        c0 = jnp.minimum(s0 - s0 % 128, L - w)
        c0 = pl.multiple_of(c0, 128)
        return x_hbm.at[pl.ds(b0, 8), pl.ds(c0, w)], c0

    def out_tile(k, bl, it):
        j0 = pl.multiple_of(j0_of(k), 8)
        return out_hbm.at[b0 + bl, pl.ds(j0, R), pl.ds(128 * it, 128)]

    def start_in(k, d, w):
        src, _ = in_slice(k, w)
        pltpu.async_copy(src, in_bufs[d].at[:, pl.ds(0, w)], in_sems[d])

    start_in(0, 0, TIER_W[0])

    for t in range(4):
        wt = TIER_W[t]

        @pl.loop(8 * t, 8 * t + 8, step=2)
        def block_loop(k0):
            for d in range(2):
                k = k0 + d
                in_buf = in_bufs[d]

                @pl.when(k + 1 < 8 * t + 8)
                def _():
                    start_in(k + 1, 1 - d, wt)

                src, c0 = in_slice(k, wt)
                pltpu.make_async_copy(src, in_buf.at[:, pl.ds(0, wt)], in_sems[d]).wait()
                j0 = j0_of(k)

                @pl.loop(0, BG)
                def batch_loop(bl):
                    slot = bl % NSLOT
                    slab = out_scr.at[slot]

                    @pl.when(k >= 1)
                    def _():
                        for _i in range(4):
                            pltpu.make_async_copy(
                                slab.at[0], out_tile(k, bl, 0), out_sems.at[slot]
                            ).wait()

                    @pl.loop(0, R)
                    def row_loop(r):
                        j = j0 + r
                        off = _start(j) - c0
                        s = off % 16
                        a0 = pl.multiple_of(off - s, 16)
                        perm = (iota + s) % 16
                        low = iota < (16 - s)
                        prev = _perm16(in_buf[bl, pl.ds(a0 + t * 128, 16)], perm)
                        for v in range(8 * t, N // 16):
                            nxt = _perm16(in_buf[bl, pl.ds(a0 + v * 16 + 16, 16)], perm)
                            vals = jnp.where(low, prev, nxt)
                            col = iota + (v * 16)
                            slab[v // 8, r, pl.ds((v * 16) % 128, 16)] = jnp.where(
                                col >= j, vals, 0.0
                            )
                            prev = nxt

                    for it in range(t):
                        pltpu.async_copy(zeros, out_tile(k, bl, it), out_sems.at[slot])
                    for it in range(t, 4):
                        pltpu.async_copy(
                            slab.at[it], out_tile(k, bl, it), out_sems.at[slot]
                        )

        if t < 3:
            start_in(8 * (t + 1), 0, TIER_W[t + 1])

    for bl in range(BG):
        slot = bl % NSLOT
        for _i in range(4):
            pltpu.make_async_copy(
                out_scr.at[slot, 0], out_tile(K - 1, bl, 0), out_sems.at[slot]
            ).wait()


@functools.partial(jax.jit, static_argnames=("interpret",))
def _unpack_triu(x, interpret=False):
    mesh = plsc.VectorSubcoreMesh(
        core_axis_name="c", subcore_axis_name="s", num_cores=NC, num_subcores=NS
    )
    f = pl.kernel(
        _body,
        out_type=jax.ShapeDtypeStruct((B, N, N), jnp.float32),
        mesh=mesh,
        scratch_types=[
            pltpu.VMEM((BG, WMAX + 16), jnp.float32),
            pltpu.VMEM((BG, WMAX + 16), jnp.float32),
            pltpu.VMEM((NSLOT, 4, R, 128), jnp.float32),
            pltpu.VMEM((R, 128), jnp.float32),
            pltpu.SemaphoreType.DMA,
            pltpu.SemaphoreType.DMA,
            pltpu.SemaphoreType.DMA((NSLOT,)),
        ],
        compiler_params=pltpu.CompilerParams(use_tc_tiling_on_sc=True),
        interpret=interpret,
    )
    return f(x)


def kernel(x, idx):
    return _unpack_triu(x)


# speedup vs baseline: 2.9411x; 2.0344x over previous
import functools

import jax
import jax.numpy as jnp
from jax import lax
from jax.experimental import pallas as pl
from jax.experimental.pallas import tpu as pltpu
from jax.experimental.pallas import tpu_sc as plsc

N = 512
B = 128
L = N * (N + 1) // 2
NC = 2
NS = 16
NW = NC * NS
R = 8
BG = 8
K = 32
NSLOT = 8
WMAX = 4224
TIER_W = (4224, 3328, 2432, 1536)


def _start(j):
    return j * N - (j * (j - 1)) // 2 - j


_PERM_DNUMS = lax.GatherDimensionNumbers(
    offset_dims=(), collapsed_slice_dims=(0,), start_index_map=(0,)
)


def _perm16(vec, idx16):
    return lax.gather(
        vec, idx16[:, None], _PERM_DNUMS, (1,),
        mode=lax.GatherScatterMode.PROMISE_IN_BOUNDS,
    )


def _body(x_hbm, out_hbm, in0, in1, out_scr, zeros, sin0, sin1, out_sems):
    wid = lax.axis_index("s") * NC + lax.axis_index("c")
    bg = wid // 2
    h = wid % 2
    b0 = pl.multiple_of(bg * BG, 8)
    iota = lax.iota(jnp.int32, 16)
    in_bufs = (in0, in1)
    in_sems = (sin0, sin1)
    zvec = jnp.zeros((16,), jnp.float32)

    @pl.loop(0, 8)
    def zero_init(r):
        for v in range(8):
            zeros[r, pl.ds(v * 16, 16)] = zvec

    def j0_of(k):
        return 16 * k + 8 * h

    def in_slice(k, w):
        s0 = _start(j0_of(k))
        c0 = jnp.minimum(s0 - s0 % 128, L - w)
        c0 = pl.multiple_of(c0, 128)
        return x_hbm.at[pl.ds(b0, 8), pl.ds(c0, w)], c0

    def out_tile(k, bl, it):
        j0 = pl.multiple_of(j0_of(k), 8)
        return out_hbm.at[b0 + bl, pl.ds(j0, R), pl.ds(128 * it, 128)]

    def start_in(k, d, w):
        src, _ = in_slice(k, w)
        pltpu.async_copy(src, in_bufs[d].at[:, pl.ds(0, w)], in_sems[d])

    start_in(0, 0, TIER_W[0])

    for t in range(4):
        wt = TIER_W[t]

        @pl.loop(8 * t, 8 * t + 8, step=2)
        def block_loop(k0):
            for d in range(2):
                k = k0 + d
                in_buf = in_bufs[d]

                @pl.when(k + 1 < 8 * t + 8)
                def _():
                    start_in(k + 1, 1 - d, wt)

                src, c0 = in_slice(k, wt)
                pltpu.make_async_copy(src, in_buf.at[:, pl.ds(0, wt)], in_sems[d]).wait()
                j0 = j0_of(k)

                @pl.loop(0, BG)
                def batch_loop(bl):
                    slot = bl % NSLOT
                    slab = out_scr.at[slot]

                    @pl.when(k >= 1)
                    def _():
                        for _i in range(4):
                            pltpu.make_async_copy(
                                slab.at[0], out_tile(k, bl, 0), out_sems.at[slot]
                            ).wait()

                    @plsc.parallel_loop(0, R, unroll=2)
                    def row_loop(r):
                        j = j0 + r
                        off = _start(j) - c0
                        s = off % 16
                        a0 = pl.multiple_of(off - s, 16)
                        perm = (iota + s) % 16
                        low = iota < (16 - s)
                        prev = _perm16(in_buf[bl, pl.ds(a0 + t * 128, 16)], perm)
                        for v in range(8 * t, N // 16):
                            nxt = _perm16(in_buf[bl, pl.ds(a0 + v * 16 + 16, 16)], perm)
                            vals = jnp.where(low, prev, nxt)
                            col = iota + (v * 16)
                            slab[v // 8, r, pl.ds((v * 16) % 128, 16)] = jnp.where(
                                col >= j, vals, 0.0
                            )
                            prev = nxt

                    for it in range(t):
                        pltpu.async_copy(zeros, out_tile(k, bl, it), out_sems.at[slot])
                    for it in range(t, 4):
                        pltpu.async_copy(
                            slab.at[it], out_tile(k, bl, it), out_sems.at[slot]
                        )

        if t < 3:
            start_in(8 * (t + 1), 0, TIER_W[t + 1])

    for bl in range(BG):
        slot = bl % NSLOT
        for _i in range(4):
            pltpu.make_async_copy(
                out_scr.at[slot, 0], out_tile(K - 1, bl, 0), out_sems.at[slot]
            ).wait()


@functools.partial(jax.jit, static_argnames=("interpret",))
def _unpack_triu(x, interpret=False):
    mesh = plsc.VectorSubcoreMesh(
        core_axis_name="c", subcore_axis_name="s", num_cores=NC, num_subcores=NS
    )
    f = pl.kernel(
        _body,
        out_type=jax.ShapeDtypeStruct((B, N, N), jnp.float32),
        mesh=mesh,
        scratch_types=[
            pltpu.VMEM((BG, WMAX + 16), jnp.float32),
            pltpu.VMEM((BG, WMAX + 16), jnp.float32),
            pltpu.VMEM((NSLOT, 4, R, 128), jnp.float32),
            pltpu.VMEM((R, 128), jnp.float32),
            pltpu.SemaphoreType.DMA,
            pltpu.SemaphoreType.DMA,
            pltpu.SemaphoreType.DMA((NSLOT,)),
        ],
        compiler_params=pltpu.CompilerParams(use_tc_tiling_on_sc=True),
        interpret=interpret,
    )
    return f(x)


def kernel(x, idx):
    return _unpack_triu(x)


# speedup vs baseline: 3.5947x; 1.2222x over previous
import functools

import jax
import jax.numpy as jnp
from jax import lax
from jax.experimental import pallas as pl
from jax.experimental.pallas import tpu as pltpu
from jax.experimental.pallas import tpu_sc as plsc

N = 512
B = 128
L = N * (N + 1) // 2
NC = 2
NS = 16
NW = NC * NS
R = 8
BG = 8
K = 32
NSLOT = 8
WMAX = 4224
TIER_W = (4224, 3328, 2432, 1536)


def _start(j):
    return j * N - (j * (j - 1)) // 2 - j


_PERM_DNUMS = lax.GatherDimensionNumbers(
    offset_dims=(), collapsed_slice_dims=(0,), start_index_map=(0,)
)


def _perm16(vec, idx16):
    return lax.gather(
        vec, idx16[:, None], _PERM_DNUMS, (1,),
        mode=lax.GatherScatterMode.PROMISE_IN_BOUNDS,
    )


def _body(x_hbm, out_hbm, in0, in1, out_scr, zeros, sin0, sin1, out_sems):
    wid = lax.axis_index("s") * NC + lax.axis_index("c")
    bg = wid // 2
    h = wid % 2
    b0 = pl.multiple_of(bg * BG, 8)
    iota = lax.iota(jnp.int32, 16)
    in_bufs = (in0, in1)
    in_sems = (sin0, sin1)
    zvec = jnp.zeros((16,), jnp.float32)

    @pl.loop(0, 8)
    def zero_init(r):
        for v in range(8):
            zeros[r, pl.ds(v * 16, 16)] = zvec

    def j0_of(k):
        return 16 * k + 8 * h

    def in_slice(k, w):
        s0 = _start(j0_of(k))
        c0 = jnp.minimum(s0 - s0 % 128, L - w)
        c0 = pl.multiple_of(c0, 128)
        return x_hbm.at[pl.ds(b0, 8), pl.ds(c0, w)], c0

    def out_tile(k, bl, it):
        j0 = pl.multiple_of(j0_of(k), 8)
        return out_hbm.at[b0 + bl, pl.ds(j0, R), pl.ds(128 * it, 128)]

    def start_in(k, d, w):
        src, _ = in_slice(k, w)
        pltpu.async_copy(src, in_bufs[d].at[:, pl.ds(0, w)], in_sems[d])

    start_in(0, 0, TIER_W[0])

    for t in range(4):
        wt = TIER_W[t]

        @pl.loop(8 * t, 8 * t + 8, step=2)
        def block_loop(k0):
            for d in range(2):
                k = k0 + d
                in_buf = in_bufs[d]

                @pl.when(k + 1 < 8 * t + 8)
                def _():
                    start_in(k + 1, 1 - d, wt)

                src, c0 = in_slice(k, wt)
                pltpu.make_async_copy(src, in_buf.at[:, pl.ds(0, wt)], in_sems[d]).wait()
                j0 = j0_of(k)

                @pl.loop(0, BG)
                def batch_loop(bl):
                    slot = bl % NSLOT
                    slab = out_scr.at[slot]

                    @pl.when(k >= 1)
                    def _():
                        for _i in range(4):
                            pltpu.make_async_copy(
                                slab.at[0], out_tile(k, bl, 0), out_sems.at[slot]
                            ).wait()

                    @plsc.parallel_loop(0, R, unroll=4)
                    def row_loop(r):
                        j = j0 + r
                        off = _start(j) - c0
                        s = off % 16
                        a0 = pl.multiple_of(off - s, 16)
                        perm = (iota + s) % 16
                        low = iota < (16 - s)
                        prev = _perm16(in_buf[bl, pl.ds(a0 + t * 128, 16)], perm)
                        for v in range(8 * t, N // 16):
                            nxt = _perm16(in_buf[bl, pl.ds(a0 + v * 16 + 16, 16)], perm)
                            vals = jnp.where(low, prev, nxt)
                            col = iota + (v * 16)
                            slab[v // 8, r, pl.ds((v * 16) % 128, 16)] = jnp.where(
                                col >= j, vals, 0.0
                            )
                            prev = nxt

                    for it in range(t):
                        pltpu.async_copy(zeros, out_tile(k, bl, it), out_sems.at[slot])
                    for it in range(t, 4):
                        pltpu.async_copy(
                            slab.at[it], out_tile(k, bl, it), out_sems.at[slot]
                        )

        if t < 3:
            start_in(8 * (t + 1), 0, TIER_W[t + 1])

    for bl in range(BG):
        slot = bl % NSLOT
        for _i in range(4):
            pltpu.make_async_copy(
                out_scr.at[slot, 0], out_tile(K - 1, bl, 0), out_sems.at[slot]
            ).wait()


@functools.partial(jax.jit, static_argnames=("interpret",))
def _unpack_triu(x, interpret=False):
    mesh = plsc.VectorSubcoreMesh(
        core_axis_name="c", subcore_axis_name="s", num_cores=NC, num_subcores=NS
    )
    f = pl.kernel(
        _body,
        out_type=jax.ShapeDtypeStruct((B, N, N), jnp.float32),
        mesh=mesh,
        scratch_types=[
            pltpu.VMEM((BG, WMAX + 16), jnp.float32),
            pltpu.VMEM((BG, WMAX + 16), jnp.float32),
            pltpu.VMEM((NSLOT, 4, R, 128), jnp.float32),
            pltpu.VMEM((R, 128), jnp.float32),
            pltpu.SemaphoreType.DMA,
            pltpu.SemaphoreType.DMA,
            pltpu.SemaphoreType.DMA((NSLOT,)),
        ],
        compiler_params=pltpu.CompilerParams(use_tc_tiling_on_sc=True),
        interpret=interpret,
    )
    return f(x)


def kernel(x, idx):
    return _unpack_triu(x)
